# baseline (speedup 1.0000x reference)
import functools

import jax
import jax.numpy as jnp
from jax import lax
from jax.experimental import pallas as pl
from jax.experimental.pallas import tpu as pltpu
from jax.experimental.pallas import tpu_sc as plsc

_OFFSETS = [(-1, -1), (-1, 0), (-1, 1), (0, 1), (1, 1), (1, 0), (1, -1), (0, -1)]

_NC = 2
_NS = 16
_L = 16


def _code_body(x_ref, o_ref, *, H, W):
    r = x_ref[0, 0]
    g = x_ref[0, 1]
    b = x_ref[0, 2]
    gray = 0.2989 * r + 0.587 * g + 0.114 * b

    gp = jnp.concatenate([gray[:1], gray, gray[-1:]], axis=0)
    gp = jnp.concatenate([gp[:, :1], gp, gp[:, -1:]], axis=1)

    code = jnp.zeros((H, W), jnp.int32)
    for k, (dy, dx) in enumerate(_OFFSETS):
        neigh = jax.lax.slice(gp, (1 + dy, 1 + dx), (1 + dy + H, 1 + dx + W))
        code = code + (1 << k) * (neigh >= gray).astype(jnp.int32)
    q = H // 4
    packed = (code[0:q]
              | (code[q:2 * q] << 8)
              | (code[2 * q:3 * q] << 16)
              | (code[3 * q:4 * q] << 24))
    o_ref[0] = packed


def _make_sc_hist(n_img, H, W, rows, spi):
    n_chunks = H // rows
    cpp = n_chunks // spi
    mesh = plsc.VectorSubcoreMesh(core_axis_name="c", subcore_axis_name="s")

    @functools.partial(
        pl.kernel,
        mesh=mesh,
        compiler_params=pltpu.CompilerParams(needs_layout_passes=False),
        out_type=jax.ShapeDtypeStruct((n_img, spi, 256), jnp.float32),
        scratch_types=[
            pltpu.VMEM((rows, W), jnp.int32),
            pltpu.VMEM((rows, W), jnp.int32),
            pltpu.VMEM((_L * 256,), jnp.float32),
            pltpu.VMEM((_L * 256,), jnp.float32),
            pltpu.VMEM((_L * 256,), jnp.float32),
            pltpu.VMEM((_L * 256,), jnp.float32),
            pltpu.VMEM((256,), jnp.float32),
            pltpu.SemaphoreType.DMA,
            pltpu.SemaphoreType.DMA,
        ],
    )
    def sc_hist(codes_hbm, out_hbm, buf0, buf1, h0, h1, h2, h3, out_v,
                sem0, sem1):
        hists = [h0, h1, h2, h3]
        wid = lax.axis_index("s") * _NC + lax.axis_index("c")
        img = wid // spi
        part = wid % spi

        def zero_step(i, _):
            z = jnp.zeros((_L,), jnp.float32)
            for h in hists:
                h[pl.ds(i * _L, _L)] = z
            return 0
        lax.fori_loop(0, (_L * 256) // _L, zero_step, 0)

        lane = lax.broadcasted_iota(jnp.int32, (_L,), 0)
        ones = jnp.ones((_L,), jnp.float32)

        bufs = [buf0, buf1]
        sems = [sem0, sem1]

        def start(c, slot):
            return pltpu.async_copy(
                codes_hbm.at[img, pl.ds((part * cpp + c) * rows, rows), :],
                bufs[slot], sems[slot])

        gpr = W // _L

        def consume(slot):
            buf = bufs[slot]

            @plsc.parallel_loop(0, rows * gpr, 1, unroll=4)
            def _(g):
                i = g // gpr
                j = g % gpr
                w = buf[i, pl.ds(j * _L, _L)]
                for k in range(4):
                    c = (w >> (8 * k)) & 255
                    plsc.addupdate_scatter(hists[k], [c * _L + lane], ones)

        cp = start(0, 0)
        for c in range(cpp):
            slot = c % 2
            cp.wait()
            if c + 1 < cpp:
                nxt = start(c + 1, 1 - slot)
            consume(slot)
            if c + 1 < cpp:
                cp = nxt

        def merge_step(i, _):
            sl = pl.ds(i * _L, _L)
            h0[sl] = (h0[sl] + h1[sl]) + (h2[sl] + h3[sl])
            return 0
        lax.fori_loop(0, (_L * 256) // _L, merge_step, 0)

        def fold_step(t, _):
            base = (t * _L + lane) * _L
            acc = jnp.zeros((_L,), jnp.float32)
            for l in range(_L):
                acc = acc + plsc.load_gather(h0, [base + l])
            out_v[pl.ds(t * _L, _L)] = acc
            return 0
        lax.fori_loop(0, 256 // _L, fold_step, 0)

        pltpu.sync_copy(out_v, out_hbm.at[img, part])

    return sc_hist


def _reduce_body(h_ref, o_ref):
    s = jnp.sum(h_ref[0], axis=0, keepdims=True)
    norm = jnp.sqrt(jnp.sum(s * s))
    o_ref[0] = s / (norm + 1e-6)


def kernel(x):
    bs, _, H, W = x.shape
    hw = H // 4
    rows = 16 if hw % 16 == 0 else hw
    n_chunks = hw // rows

    n_halves = 2 if (bs % 2 == 0 and bs >= 2) else 1
    bsh = bs // n_halves
    spi = max(1, (_NC * _NS) // bsh)
    if n_chunks % spi != 0 or (_NC * _NS) % bsh != 0:
        spi = 1
    sc_hist = _make_sc_hist(bsh, hw, W, rows, spi)

    stencil = pl.pallas_call(
        functools.partial(_code_body, H=H, W=W),
        grid=(bsh,),
        in_specs=[pl.BlockSpec((1, 3, H, W), lambda i: (i, 0, 0, 0))],
        out_specs=pl.BlockSpec((1, hw, W), lambda i: (i, 0, 0)),
        out_shape=jax.ShapeDtypeStruct((bsh, hw, W), jnp.int32),
    )

    reduce_fn = pl.pallas_call(
        _reduce_body,
        grid=(bsh,),
        in_specs=[pl.BlockSpec((1, spi, 256), lambda i: (i, 0, 0))],
        out_specs=pl.BlockSpec((1, 1, 256), lambda i: (i, 0, 0)),
        out_shape=jax.ShapeDtypeStruct((bsh, 1, 256), jnp.float32),
    )

    outs = []
    for h in range(n_halves):
        xh = jax.lax.slice_in_dim(x, h * bsh, (h + 1) * bsh, axis=0)
        codes = stencil(xh)
        hist = sc_hist(codes)
        outs.append(reduce_fn(hist).reshape(bsh, 256))
    return jnp.concatenate(outs, axis=0) if n_halves > 1 else outs[0]


# speedup vs baseline: 1.2979x; 1.2979x over previous
import functools

import jax
import jax.numpy as jnp
from jax import lax
from jax.experimental import pallas as pl
from jax.experimental.pallas import tpu as pltpu
from jax.experimental.pallas import tpu_sc as plsc

_OFFSETS = [(-1, -1), (-1, 0), (-1, 1), (0, 1), (1, 1), (1, 0), (1, -1), (0, -1)]

_NC = 2
_NS = 16
_L = 16


def _code_body(x_ref, o_ref, *, H, W):
    r = x_ref[0, 0]
    g = x_ref[0, 1]
    b = x_ref[0, 2]
    gray = 0.2989 * r + 0.587 * g + 0.114 * b

    gp = jnp.concatenate([gray[:1], gray, gray[-1:]], axis=0)
    gp = jnp.concatenate([gp[:, :1], gp, gp[:, -1:]], axis=1)

    code = jnp.zeros((H, W), jnp.int32)
    for k, (dy, dx) in enumerate(_OFFSETS):
        neigh = jax.lax.slice(gp, (1 + dy, 1 + dx), (1 + dy + H, 1 + dx + W))
        code = code + (1 << k) * (neigh >= gray).astype(jnp.int32)
    q = H // 4
    packed = (code[0:q]
              | (code[q:2 * q] << 8)
              | (code[2 * q:3 * q] << 16)
              | (code[3 * q:4 * q] << 24))
    o_ref[0] = packed


def _make_sc_hist(n_img, H, W, rows, spi):
    n_chunks = H // rows
    cpp = n_chunks // spi
    mesh = plsc.VectorSubcoreMesh(core_axis_name="c", subcore_axis_name="s")

    @functools.partial(
        pl.kernel,
        mesh=mesh,
        compiler_params=pltpu.CompilerParams(needs_layout_passes=False),
        out_type=jax.ShapeDtypeStruct((n_img, spi, 256), jnp.float32),
        scratch_types=[
            pltpu.VMEM((rows, W), jnp.int32),
            pltpu.VMEM((rows, W), jnp.int32),
            pltpu.VMEM((_L * 256,), jnp.float32),
            pltpu.VMEM((_L * 256,), jnp.float32),
            pltpu.VMEM((_L * 256,), jnp.float32),
            pltpu.VMEM((_L * 256,), jnp.float32),
            pltpu.VMEM((256,), jnp.float32),
            pltpu.SemaphoreType.DMA,
            pltpu.SemaphoreType.DMA,
        ],
    )
    def sc_hist(codes_hbm, out_hbm, buf0, buf1, h0, h1, h2, h3, out_v,
                sem0, sem1):
        hists = [h0, h1, h2, h3]
        wid = lax.axis_index("s") * _NC + lax.axis_index("c")
        img = wid // spi
        part = wid % spi

        def zero_step(i, _):
            z = jnp.zeros((_L,), jnp.float32)
            for h in hists:
                h[pl.ds(i * _L, _L)] = z
            return 0
        lax.fori_loop(0, (_L * 256) // _L, zero_step, 0)

        lane = lax.broadcasted_iota(jnp.int32, (_L,), 0)
        ones = jnp.ones((_L,), jnp.float32)

        bufs = [buf0, buf1]
        sems = [sem0, sem1]

        def start(c, slot):
            return pltpu.async_copy(
                codes_hbm.at[img, pl.ds((part * cpp + c) * rows, rows), :],
                bufs[slot], sems[slot])

        gpr = W // _L

        def consume(slot):
            buf = bufs[slot]

            @plsc.parallel_loop(0, rows * gpr, 1, unroll=4)
            def _(g):
                i = g // gpr
                j = g % gpr
                w = buf[i, pl.ds(j * _L, _L)]
                for k in range(4):
                    c = (w >> (8 * k)) & 255
                    plsc.addupdate_scatter(hists[k], [c * _L + lane], ones)

        cp = start(0, 0)
        for c in range(cpp):
            slot = c % 2
            cp.wait()
            if c + 1 < cpp:
                nxt = start(c + 1, 1 - slot)
            consume(slot)
            if c + 1 < cpp:
                cp = nxt

        def merge_step(i, _):
            sl = pl.ds(i * _L, _L)
            h0[sl] = (h0[sl] + h1[sl]) + (h2[sl] + h3[sl])
            return 0
        lax.fori_loop(0, (_L * 256) // _L, merge_step, 0)

        def fold_step(t, _):
            base = (t * _L + lane) * _L
            acc = jnp.zeros((_L,), jnp.float32)
            for l in range(_L):
                acc = acc + plsc.load_gather(h0, [base + l])
            out_v[pl.ds(t * _L, _L)] = acc
            return 0
        lax.fori_loop(0, 256 // _L, fold_step, 0)

        pltpu.sync_copy(out_v, out_hbm.at[img, part])

    return sc_hist


def _reduce_body(h_ref, o_ref):
    s = jnp.sum(h_ref[0], axis=0, keepdims=True)
    norm = jnp.sqrt(jnp.sum(s * s))
    o_ref[0] = s / (norm + 1e-6)


def kernel(x):
    bs, _, H, W = x.shape
    hw = H // 4
    rows = 16 if hw % 16 == 0 else hw
    n_chunks = hw // rows

    n_halves = 1
    bsh = bs // n_halves
    spi = max(1, (_NC * _NS) // bsh)
    if n_chunks % spi != 0 or (_NC * _NS) % bsh != 0:
        spi = 1
    sc_hist = _make_sc_hist(bsh, hw, W, rows, spi)

    stencil = pl.pallas_call(
        functools.partial(_code_body, H=H, W=W),
        grid=(bsh,),
        in_specs=[pl.BlockSpec((1, 3, H, W), lambda i: (i, 0, 0, 0))],
        out_specs=pl.BlockSpec((1, hw, W), lambda i: (i, 0, 0)),
        out_shape=jax.ShapeDtypeStruct((bsh, hw, W), jnp.int32),
    )

    reduce_fn = pl.pallas_call(
        _reduce_body,
        grid=(bsh,),
        in_specs=[pl.BlockSpec((1, spi, 256), lambda i: (i, 0, 0))],
        out_specs=pl.BlockSpec((1, 1, 256), lambda i: (i, 0, 0)),
        out_shape=jax.ShapeDtypeStruct((bsh, 1, 256), jnp.float32),
    )

    outs = []
    for h in range(n_halves):
        xh = jax.lax.slice_in_dim(x, h * bsh, (h + 1) * bsh, axis=0)
        codes = stencil(xh)
        hist = sc_hist(codes)
        outs.append(reduce_fn(hist).reshape(bsh, 256))
    return jnp.concatenate(outs, axis=0) if n_halves > 1 else outs[0]
